# Initial kernel scaffold; baseline (speedup 1.0000x reference)
import jax
import jax.numpy as jnp
from jax.experimental import pallas as pl


def kernel(x, edge_index_r0, edge_index_r1, edge_index_r2, W1, b1, W2, b2, Wc, bc):
    raise NotImplementedError("write your pallas kernel here")


# speedup vs baseline: 1.8336x; 1.8336x over previous
import functools

import jax
import jax.numpy as jnp
from jax import lax
from jax.experimental import pallas as pl
from jax.experimental.pallas import tpu as pltpu
from jax.experimental.pallas import tpu_sc as plsc

_N = 50000
_NP = 50048
_D = 128
_H = 128
_C = 10
_E = 200000
_R = 3

_NC = 2
_NS = 16
_B = 128
_CHUNKS = 98
_EPAD = _NS * _CHUNKS * _B
_RPT = _NP // _NS
_ZROWS = 184
_NZ = 17
_FC = 16
_NCH = _H // _FC
_BN = 3128


def _sc_mesh():
    return plsc.VectorSubcoreMesh(
        core_axis_name="c", subcore_axis_name="s",
        num_cores=_NC, num_subcores=_NS)


_SC_PARAMS = pltpu.CompilerParams(use_tc_tiling_on_sc=False)


def _sc_bincount(idx, ones16, zeros16):
    @functools.partial(
        pl.kernel,
        out_type=jax.ShapeDtypeStruct((_R, 2, _NP, 16), jnp.float32),
        mesh=_sc_mesh(),
        scratch_types=[
            pltpu.VMEM((_CHUNKS, _B), jnp.int32),
            pltpu.VMEM((_B, 16), jnp.float32),
            pltpu.VMEM((_ZROWS, 16), jnp.float32),
            pltpu.VMEM_SHARED((_NP, 16), jnp.float32),
        ],
        compiler_params=_SC_PARAMS,
    )
    def k(idx_hbm, ones_hbm, z_hbm, out_hbm, eidx, ones, zbuf, acc):
        cid = lax.axis_index("c")
        sid = lax.axis_index("s")
        pltpu.sync_copy(ones_hbm, ones)
        pltpu.sync_copy(z_hbm, zbuf)
        row0 = sid * _RPT
        for r in range(_R):
            pltpu.sync_copy(idx_hbm.at[r].at[cid].at[sid], eidx)
            for z in range(_NZ):
                pltpu.sync_copy(zbuf, acc.at[pl.ds(row0 + z * _ZROWS, _ZROWS)])
            plsc.subcore_barrier()

            def body(i, carry):
                pltpu.sync_copy(ones, acc.at[eidx.at[i]], add=True)
                return carry

            lax.fori_loop(0, _CHUNKS, body, 0)
            plsc.subcore_barrier()
            pltpu.sync_copy(acc.at[pl.ds(row0, _RPT)],
                            out_hbm.at[r].at[cid].at[pl.ds(row0, _RPT)])
            plsc.subcore_barrier()

    return k(idx, ones16, zeros16)


def _sc_segsum(yt8, gidx, idx, zeros16):
    @functools.partial(
        pl.kernel,
        out_type=jax.ShapeDtypeStruct((_R, _NP, _H), jnp.float32),
        mesh=_sc_mesh(),
        scratch_types=[
            pltpu.VMEM((_CHUNKS, _B), jnp.int32),
            pltpu.VMEM((_CHUNKS, _B), jnp.int32),
            pltpu.VMEM((_B, _FC), jnp.float32),
            pltpu.VMEM((_ZROWS, _FC), jnp.float32),
            pltpu.VMEM_SHARED((_NP, _FC), jnp.float32),
            pltpu.SemaphoreType.DMA,
        ],
        compiler_params=_SC_PARAMS,
    )
    def k(yt_hbm, gidx_hbm, idx_hbm, z_hbm, out_hbm, sidx, didx, rows, zbuf, acc, sem):
        cid = lax.axis_index("c")
        sid = lax.axis_index("s")
        pltpu.sync_copy(z_hbm, zbuf)
        row0 = sid * _RPT
        for r in range(_R):
            pltpu.sync_copy(idx_hbm.at[r].at[1].at[sid], didx)
            for cc in range(_NCH // 2):
                c = cid * (_NCH // 2) + cc
                pltpu.sync_copy(gidx_hbm.at[r].at[c].at[sid], sidx)
                for z in range(_NZ):
                    pltpu.sync_copy(zbuf, acc.at[pl.ds(row0 + z * _ZROWS, _ZROWS)])
                plsc.subcore_barrier()

                def body(i, carry):
                    pltpu.async_copy(yt_hbm.at[r].at[sidx.at[i]], rows, sem).wait()
                    pltpu.sync_copy(rows, acc.at[didx.at[i]], add=True)
                    return carry

                lax.fori_loop(0, _CHUNKS, body, 0)
                plsc.subcore_barrier()
                pltpu.sync_copy(
                    acc.at[pl.ds(row0, _RPT)],
                    out_hbm.at[r].at[pl.ds(row0, _RPT), pl.ds(c * _FC, _FC)])
                plsc.subcore_barrier()

    return k(yt8, gidx, idx, zeros16)


def _norm_from_deg(d):
    return lax.rsqrt(jnp.where(d > 0.0, d, 1.0))


def _tc_linear(xh, deg_src, W):
    def body(x_ref, d_ref, w_ref, o_ref):
        x = x_ref[...]
        dd = d_ref[...]
        for r in range(_R):
            norm = _norm_from_deg(dd[r, :, 0:1])
            o_ref[r] = jnp.dot(x * norm, w_ref[r],
                               preferred_element_type=jnp.float32)

    return pl.pallas_call(
        body,
        grid=(_NP // _BN,),
        in_specs=[
            pl.BlockSpec((_BN, _D), lambda i: (i, 0)),
            pl.BlockSpec((_R, _BN, 16), lambda i: (0, i, 0)),
            pl.BlockSpec((_R, _D, _H), lambda i: (0, 0, 0)),
        ],
        out_specs=pl.BlockSpec((_R, _BN, _H), lambda i: (0, i, 0)),
        out_shape=jax.ShapeDtypeStruct((_R, _NP, _H), jnp.float32),
    )(xh, deg_src, W)


def _tc_combine_relu(agg, deg_dst, b):
    def body(a_ref, d_ref, b_ref, o_ref):
        dd = d_ref[...]
        bb = b_ref[...]
        bsum = bb[0:1] + bb[1:2] + bb[2:3]
        acc = a_ref[0] * _norm_from_deg(dd[0, :, 0:1])
        acc = acc + a_ref[1] * _norm_from_deg(dd[1, :, 0:1])
        acc = acc + a_ref[2] * _norm_from_deg(dd[2, :, 0:1])
        o_ref[...] = jnp.maximum(acc + bsum, 0.0)

    return pl.pallas_call(
        body,
        grid=(_NP // _BN,),
        in_specs=[
            pl.BlockSpec((_R, _BN, _H), lambda i: (0, i, 0)),
            pl.BlockSpec((_R, _BN, 16), lambda i: (0, i, 0)),
            pl.BlockSpec((_R, _H), lambda i: (0, 0)),
        ],
        out_specs=pl.BlockSpec((_BN, _H), lambda i: (i, 0)),
        out_shape=jax.ShapeDtypeStruct((_NP, _H), jnp.float32),
    )(agg, deg_dst, b)


def _tc_final(agg, deg_dst, b, Wcp, bcp):
    nblocks = _NP // _BN

    def body(a_ref, d_ref, b_ref, wc_ref, bc_ref, o_ref, hg):
        i = pl.program_id(0)

        @pl.when(i == 0)
        def _():
            hg[...] = jnp.zeros_like(hg)

        dd = d_ref[...]
        acc = a_ref[0] * _norm_from_deg(dd[0, :, 0:1])
        acc = acc + a_ref[1] * _norm_from_deg(dd[1, :, 0:1])
        acc = acc + a_ref[2] * _norm_from_deg(dd[2, :, 0:1])
        rid = lax.broadcasted_iota(jnp.int32, (_BN, 1), 0) + i * _BN
        acc = jnp.where(rid < _N, acc, 0.0)
        hg[...] += jnp.sum(acc, axis=0, keepdims=True)

        @pl.when(i == nblocks - 1)
        def _():
            bb = b_ref[...]
            bsum = bb[0:1] + bb[1:2] + bb[2:3]
            hgv = hg[...] * (1.0 / _N) + bsum
            logits = jnp.dot(hgv, wc_ref[...],
                             preferred_element_type=jnp.float32) + bc_ref[...]
            m = jnp.max(logits, axis=-1, keepdims=True)
            e = jnp.exp(logits - m)
            o_ref[...] = e / jnp.sum(e, axis=-1, keepdims=True)

    return pl.pallas_call(
        body,
        grid=(nblocks,),
        in_specs=[
            pl.BlockSpec((_R, _BN, _H), lambda i: (0, i, 0)),
            pl.BlockSpec((_R, _BN, 16), lambda i: (0, i, 0)),
            pl.BlockSpec((_R, _H), lambda i: (0, 0)),
            pl.BlockSpec((_H, _H), lambda i: (0, 0)),
            pl.BlockSpec((1, _H), lambda i: (0, 0)),
        ],
        out_specs=pl.BlockSpec((1, _H), lambda i: (0, 0)),
        out_shape=jax.ShapeDtypeStruct((1, _H), jnp.float32),
        scratch_shapes=[pltpu.VMEM((1, _H), jnp.float32)],
    )(agg, deg_dst, b, Wcp, bcp)


def _pad_split(v, pad_val):
    pad = _EPAD - _E
    vp = jnp.concatenate([v, jnp.full((pad,), pad_val, jnp.int32)])
    return vp.reshape(_NS, _CHUNKS, _B)


def kernel(x, edge_index_r0, edge_index_r1, edge_index_r2, W1, b1, W2, b2, Wc, bc):
    x = jnp.pad(x, ((0, _NP - _N), (0, 0)))
    edges = [e.astype(jnp.int32) for e in
             (edge_index_r0, edge_index_r1, edge_index_r2)]
    idx = jnp.stack([jnp.stack([_pad_split(e[0], 0), _pad_split(e[1], _N)])
                     for e in edges])
    gidx = jnp.stack([jnp.stack([_pad_split(e[0] * _NCH + c, c)
                                 for c in range(_NCH)]) for e in edges])
    ones16 = jnp.ones((_B, 16), jnp.float32)
    zeros16 = jnp.zeros((_ZROWS, _FC), jnp.float32)

    deg = _sc_bincount(idx, ones16, zeros16)
    deg_src = deg[:, 0]
    deg_dst = deg[:, 1]

    y1 = _tc_linear(x, deg_src, W1)
    agg1 = _sc_segsum(y1.reshape(_R, _NP * _NCH, _FC), gidx, idx, zeros16)
    h = _tc_combine_relu(agg1, deg_dst, b1)

    y2 = _tc_linear(h, deg_src, W2)
    agg2 = _sc_segsum(y2.reshape(_R, _NP * _NCH, _FC), gidx, idx, zeros16)

    Wcp = jnp.pad(Wc, ((0, 0), (0, _H - _C)))
    bcp = jnp.pad(bc, (0, _H - _C), constant_values=-1e30).reshape(1, _H)
    probs = _tc_final(agg2, deg_dst, b2, Wcp, bcp)
    return probs[0, :_C]


# speedup vs baseline: 3.0155x; 1.6446x over previous
import functools

import jax
import jax.numpy as jnp
from jax import lax
from jax.experimental import pallas as pl
from jax.experimental.pallas import tpu as pltpu
from jax.experimental.pallas import tpu_sc as plsc

_N = 50000
_NP = 50048
_D = 128
_H = 128
_C = 10
_E = 200000
_R = 3

_NC = 2
_NS = 16
_B = 128
_CHUNKS = 98
_EPAD = _NS * _CHUNKS * _B
_RPT = _NP // _NS
_ZROWS = 184
_NZ = 17
_G = 7
_FC = 16
_NCH = _H // _FC
_BN = 3128


def _sc_mesh():
    return plsc.VectorSubcoreMesh(
        core_axis_name="c", subcore_axis_name="s",
        num_cores=_NC, num_subcores=_NS)


_SC_PARAMS = pltpu.CompilerParams(use_tc_tiling_on_sc=False)


def _sc_bincount(idx, ones16, zeros16):
    @functools.partial(
        pl.kernel,
        out_type=jax.ShapeDtypeStruct((_R, 2, _NP, 16), jnp.float32),
        mesh=_sc_mesh(),
        scratch_types=[
            pltpu.VMEM((_CHUNKS, _B), jnp.int32),
            pltpu.VMEM((_B, 16), jnp.float32),
            pltpu.VMEM((_ZROWS, 16), jnp.float32),
            pltpu.VMEM_SHARED((_NP, 16), jnp.float32),
        ],
        compiler_params=_SC_PARAMS,
    )
    def k(idx_hbm, ones_hbm, z_hbm, out_hbm, eidx, ones, zbuf, acc):
        cid = lax.axis_index("c")
        sid = lax.axis_index("s")
        pltpu.sync_copy(ones_hbm, ones)
        pltpu.sync_copy(z_hbm, zbuf)
        row0 = sid * _RPT
        for r in range(_R):
            pltpu.sync_copy(idx_hbm.at[r].at[cid].at[sid], eidx)
            for z in range(_NZ):
                pltpu.sync_copy(zbuf, acc.at[pl.ds(row0 + z * _ZROWS, _ZROWS)])
            plsc.subcore_barrier()

            def body(i, carry):
                pltpu.sync_copy(ones, acc.at[eidx.at[i]], add=True)
                return carry

            lax.fori_loop(0, _CHUNKS, body, 0)
            plsc.subcore_barrier()
            pltpu.sync_copy(acc.at[pl.ds(row0, _RPT)],
                            out_hbm.at[r].at[cid].at[pl.ds(row0, _RPT)])
            plsc.subcore_barrier()

    return k(idx, ones16, zeros16)


def _sc_segsum(yt8, gidx, idx, zeros16):
    @functools.partial(
        pl.kernel,
        out_type=jax.ShapeDtypeStruct((_R, _NP, _H), jnp.float32),
        mesh=_sc_mesh(),
        scratch_types=[
            pltpu.VMEM((_CHUNKS, _B), jnp.int32),
            pltpu.VMEM((_CHUNKS, _B), jnp.int32),
            pltpu.VMEM((_G, _B, _FC), jnp.float32),
            pltpu.VMEM((_ZROWS, _FC), jnp.float32),
            pltpu.VMEM_SHARED((_NP, _FC), jnp.float32),
            pltpu.SemaphoreType.DMA,
        ],
        compiler_params=_SC_PARAMS,
    )
    def k(yt_hbm, gidx_hbm, idx_hbm, z_hbm, out_hbm, sidx, didx, rows, zbuf, acc, sem):
        cid = lax.axis_index("c")
        sid = lax.axis_index("s")
        pltpu.sync_copy(z_hbm, zbuf)
        row0 = sid * _RPT
        for r in range(_R):
            pltpu.sync_copy(idx_hbm.at[r].at[1].at[sid], didx)
            for cc in range(_NCH // 2):
                c = cid * (_NCH // 2) + cc
                pltpu.sync_copy(gidx_hbm.at[r].at[c].at[sid], sidx)
                for z in range(_NZ):
                    pltpu.sync_copy(zbuf, acc.at[pl.ds(row0 + z * _ZROWS, _ZROWS)])
                plsc.subcore_barrier()

                def body(j, carry):
                    descs = [
                        pltpu.async_copy(
                            yt_hbm.at[r].at[sidx.at[j * _G + g]],
                            rows.at[g], sem)
                        for g in range(_G)]
                    for g in range(_G):
                        descs[g].wait()
                        pltpu.sync_copy(rows.at[g],
                                        acc.at[didx.at[j * _G + g]], add=True)
                    return carry

                lax.fori_loop(0, _CHUNKS // _G, body, 0)
                plsc.subcore_barrier()
                pltpu.sync_copy(
                    acc.at[pl.ds(row0, _RPT)],
                    out_hbm.at[r].at[pl.ds(row0, _RPT), pl.ds(c * _FC, _FC)])
                plsc.subcore_barrier()

    return k(yt8, gidx, idx, zeros16)


def _norm_from_deg(d):
    return lax.rsqrt(jnp.where(d > 0.0, d, 1.0))


def _tc_linear(xh, deg_src, W):
    def body(x_ref, d_ref, w_ref, o_ref):
        x = x_ref[...]
        dd = d_ref[...]
        for r in range(_R):
            norm = _norm_from_deg(dd[r, :, 0:1])
            o_ref[r] = jnp.dot(x * norm, w_ref[r],
                               preferred_element_type=jnp.float32)

    return pl.pallas_call(
        body,
        grid=(_NP // _BN,),
        in_specs=[
            pl.BlockSpec((_BN, _D), lambda i: (i, 0)),
            pl.BlockSpec((_R, _BN, 16), lambda i: (0, i, 0)),
            pl.BlockSpec((_R, _D, _H), lambda i: (0, 0, 0)),
        ],
        out_specs=pl.BlockSpec((_R, _BN, _H), lambda i: (0, i, 0)),
        out_shape=jax.ShapeDtypeStruct((_R, _NP, _H), jnp.float32),
    )(xh, deg_src, W)


def _tc_combine_relu(agg, deg_dst, b):
    def body(a_ref, d_ref, b_ref, o_ref):
        dd = d_ref[...]
        bb = b_ref[...]
        bsum = bb[0:1] + bb[1:2] + bb[2:3]
        acc = a_ref[0] * _norm_from_deg(dd[0, :, 0:1])
        acc = acc + a_ref[1] * _norm_from_deg(dd[1, :, 0:1])
        acc = acc + a_ref[2] * _norm_from_deg(dd[2, :, 0:1])
        o_ref[...] = jnp.maximum(acc + bsum, 0.0)

    return pl.pallas_call(
        body,
        grid=(_NP // _BN,),
        in_specs=[
            pl.BlockSpec((_R, _BN, _H), lambda i: (0, i, 0)),
            pl.BlockSpec((_R, _BN, 16), lambda i: (0, i, 0)),
            pl.BlockSpec((_R, _H), lambda i: (0, 0)),
        ],
        out_specs=pl.BlockSpec((_BN, _H), lambda i: (i, 0)),
        out_shape=jax.ShapeDtypeStruct((_NP, _H), jnp.float32),
    )(agg, deg_dst, b)


def _tc_final(agg, deg_dst, b, Wcp, bcp):
    nblocks = _NP // _BN

    def body(a_ref, d_ref, b_ref, wc_ref, bc_ref, o_ref, hg):
        i = pl.program_id(0)

        @pl.when(i == 0)
        def _():
            hg[...] = jnp.zeros_like(hg)

        dd = d_ref[...]
        acc = a_ref[0] * _norm_from_deg(dd[0, :, 0:1])
        acc = acc + a_ref[1] * _norm_from_deg(dd[1, :, 0:1])
        acc = acc + a_ref[2] * _norm_from_deg(dd[2, :, 0:1])
        rid = lax.broadcasted_iota(jnp.int32, (_BN, 1), 0) + i * _BN
        acc = jnp.where(rid < _N, acc, 0.0)
        hg[...] += jnp.sum(acc, axis=0, keepdims=True)

        @pl.when(i == nblocks - 1)
        def _():
            bb = b_ref[...]
            bsum = bb[0:1] + bb[1:2] + bb[2:3]
            hgv = hg[...] * (1.0 / _N) + bsum
            logits = jnp.dot(hgv, wc_ref[...],
                             preferred_element_type=jnp.float32) + bc_ref[...]
            m = jnp.max(logits, axis=-1, keepdims=True)
            e = jnp.exp(logits - m)
            o_ref[...] = e / jnp.sum(e, axis=-1, keepdims=True)

    return pl.pallas_call(
        body,
        grid=(nblocks,),
        in_specs=[
            pl.BlockSpec((_R, _BN, _H), lambda i: (0, i, 0)),
            pl.BlockSpec((_R, _BN, 16), lambda i: (0, i, 0)),
            pl.BlockSpec((_R, _H), lambda i: (0, 0)),
            pl.BlockSpec((_H, _H), lambda i: (0, 0)),
            pl.BlockSpec((1, _H), lambda i: (0, 0)),
        ],
        out_specs=pl.BlockSpec((1, _H), lambda i: (0, 0)),
        out_shape=jax.ShapeDtypeStruct((1, _H), jnp.float32),
        scratch_shapes=[pltpu.VMEM((1, _H), jnp.float32)],
    )(agg, deg_dst, b, Wcp, bcp)


def _pad_split(v, pad_val):
    pad = _EPAD - _E
    vp = jnp.concatenate([v, jnp.full((pad,), pad_val, jnp.int32)])
    return vp.reshape(_NS, _CHUNKS, _B)


def kernel(x, edge_index_r0, edge_index_r1, edge_index_r2, W1, b1, W2, b2, Wc, bc):
    x = jnp.pad(x, ((0, _NP - _N), (0, 0)))
    edges = [e.astype(jnp.int32) for e in
             (edge_index_r0, edge_index_r1, edge_index_r2)]
    idx = jnp.stack([jnp.stack([_pad_split(e[0], 0), _pad_split(e[1], _N)])
                     for e in edges])
    gidx = jnp.stack([jnp.stack([_pad_split(e[0] * _NCH + c, c)
                                 for c in range(_NCH)]) for e in edges])
    ones16 = jnp.ones((_B, 16), jnp.float32)
    zeros16 = jnp.zeros((_ZROWS, _FC), jnp.float32)

    deg = _sc_bincount(idx, ones16, zeros16)
    deg_src = deg[:, 0]
    deg_dst = deg[:, 1]

    y1 = _tc_linear(x, deg_src, W1)
    agg1 = _sc_segsum(y1.reshape(_R, _NP * _NCH, _FC), gidx, idx, zeros16)
    h = _tc_combine_relu(agg1, deg_dst, b1)

    y2 = _tc_linear(h, deg_src, W2)
    agg2 = _sc_segsum(y2.reshape(_R, _NP * _NCH, _FC), gidx, idx, zeros16)

    Wcp = jnp.pad(Wc, ((0, 0), (0, _H - _C)))
    bcp = jnp.pad(bc, (0, _H - _C), constant_values=-1e30).reshape(1, _H)
    probs = _tc_final(agg2, deg_dst, b2, Wcp, bcp)
    return probs[0, :_C]
